# baseline (speedup 1.0000x reference)
import functools

import jax
import jax.numpy as jnp
from jax import lax
from jax.experimental import pallas as pl
from jax.experimental.pallas import tpu as pltpu
from jax.experimental.pallas import tpu_sc as plsc

B = 16384
D = 64
CHUNK = 32


def _gather_concat(uid, rid, ing, ut3, rt3, it3):
    info = plsc.get_sparse_core_info()
    nc, ns = info.num_cores, info.num_subcores
    nw = nc * ns
    bpw = B // nw
    nch = bpw // CHUNK
    mesh = plsc.VectorSubcoreMesh(core_axis_name="c", subcore_axis_name="s")

    @functools.partial(
        pl.kernel,
        mesh=mesh,
        out_type=jax.ShapeDtypeStruct((B, 3 * D), jnp.float32),
        scratch_types=[
            pltpu.SMEM((bpw,), jnp.int32),
            pltpu.SMEM((bpw,), jnp.int32),
            pltpu.SMEM((bpw,), jnp.int32),
            pltpu.VMEM_SHARED((B,), jnp.int32),
            pltpu.VMEM((CHUNK, 8, D), jnp.float32),
            pltpu.VMEM((CHUNK, 8, D), jnp.float32),
            pltpu.VMEM((CHUNK, 3 * D), jnp.float32),
            pltpu.SemaphoreType.DMA,
            pltpu.SemaphoreType.DMA,
        ],
    )
    def k(uid_h, rid_h, ing_h, ut_h, rt_h, it_h, out_h,
          smem_u, smem_r, smem_g, sp, tb0, tb1, asm, sem0, sem1):
        wid = lax.axis_index("s") * nc + lax.axis_index("c")
        base = wid * bpw
        bsl = pl.ds(base, bpw)

        for ix_h, smem in ((uid_h, smem_u), (rid_h, smem_r), (ing_h, smem_g)):
            pltpu.sync_copy(ix_h.at[bsl], sp.at[bsl])
            pltpu.sync_copy(sp.at[bsl], smem)

        tabs = (ut_h, rt_h, it_h)
        smems = (smem_u, smem_r, smem_g)
        tbufs = (tb0, tb1)
        sems = (sem0, sem1)

        def fire(s):
            k_, t = divmod(s, 3)
            tab, smem = tabs[t], smems[t]
            buf, sem = tbufs[s % 2], sems[s % 2]
            off = k_ * CHUNK

            def body(i, carry):
                blk = smem[off + i] >> 3
                pltpu.async_copy(tab.at[blk], buf.at[i], sem)
                return carry

            lax.fori_loop(0, CHUNK, body, 0, unroll=8)

        def drain_and_select(s):
            k_, t = divmod(s, 3)
            tab, smem = tabs[t], smems[t]
            buf, sem = tbufs[s % 2], sems[s % 2]
            pltpu.make_async_copy(tab.at[pl.ds(0, CHUNK)], buf, sem).wait()
            off = k_ * CHUNK

            def body(i, carry):
                j = smem[off + i] & 7
                for cg in range(D // 16):
                    csl = pl.ds(cg * 16, 16)
                    asm[i, pl.ds(t * D + cg * 16, 16)] = buf[i, j, csl]
                return carry

            lax.fori_loop(0, CHUNK, body, 0)

        nsteps = nch * 3
        fire(0)
        for s in range(nsteps):
            if s + 1 < nsteps:
                fire(s + 1)
            drain_and_select(s)
            k_, t = divmod(s, 3)
            if t == 2:
                pltpu.sync_copy(
                    asm, out_h.at[pl.ds(base + k_ * CHUNK, CHUNK)])

    return k(uid, rid, ing, ut3, rt3, it3)


def kernel(uid, rid, ing, user_table, recipe_table, ingredient_table):
    ut3 = user_table.reshape(-1, 8, D)
    rt3 = recipe_table.reshape(-1, 8, D)
    it3 = ingredient_table.reshape(-1, 8, D)
    return _gather_concat(uid, rid, ing, ut3, rt3, it3)


# speedup vs baseline: 1.0115x; 1.0115x over previous
import functools

import jax
import jax.numpy as jnp
from jax import lax
from jax.experimental import pallas as pl
from jax.experimental.pallas import tpu as pltpu
from jax.experimental.pallas import tpu_sc as plsc

B = 16384
D = 64
CHUNK = 32


def _gather_concat(uid, rid, ing, ut3, rt3, it3):
    info = plsc.get_sparse_core_info()
    nc, ns = info.num_cores, info.num_subcores
    nw = nc * ns
    bpw = B // nw
    nch = bpw // CHUNK
    mesh = plsc.VectorSubcoreMesh(core_axis_name="c", subcore_axis_name="s")

    @functools.partial(
        pl.kernel,
        mesh=mesh,
        out_type=jax.ShapeDtypeStruct((B, 3 * D), jnp.float32),
        scratch_types=[
            pltpu.SMEM((bpw,), jnp.int32),
            pltpu.SMEM((bpw,), jnp.int32),
            pltpu.SMEM((bpw,), jnp.int32),
            pltpu.VMEM_SHARED((B,), jnp.int32),
            pltpu.VMEM((CHUNK, 8, D), jnp.float32),
            pltpu.VMEM((CHUNK, 8, D), jnp.float32),
            pltpu.VMEM((CHUNK, 3 * D), jnp.float32),
            pltpu.SemaphoreType.DMA,
            pltpu.SemaphoreType.DMA,
            pltpu.SemaphoreType.DMA,
            pltpu.SemaphoreType.DMA,
        ],
    )
    def k(uid_h, rid_h, ing_h, ut_h, rt_h, it_h, out_h,
          smem_u, smem_r, smem_g, sp, tb0, tb1, asm,
          sem0, sem1, sem2, sem3):
        wid = lax.axis_index("s") * nc + lax.axis_index("c")
        base = wid * bpw
        bsl = pl.ds(base, bpw)

        for ix_h, smem in ((uid_h, smem_u), (rid_h, smem_r), (ing_h, smem_g)):
            pltpu.sync_copy(ix_h.at[bsl], sp.at[bsl])
            pltpu.sync_copy(sp.at[bsl], smem)

        tabs = (ut_h, rt_h, it_h)
        smems = (smem_u, smem_r, smem_g)
        tbufs = (tb0, tb1)
        sems = ((sem0, sem1), (sem2, sem3))
        half = CHUNK // 2

        def fire(s):
            k_, t = divmod(s, 3)
            tab, smem = tabs[t], smems[t]
            buf = tbufs[s % 2]
            sa, sb = sems[s % 2]
            off = k_ * CHUNK

            def body(i, carry):
                blk = smem[off + i] >> 3
                pltpu.async_copy(tab.at[blk], buf.at[i], sa)
                blk2 = smem[off + half + i] >> 3
                pltpu.async_copy(tab.at[blk2], buf.at[half + i], sb)
                return carry

            lax.fori_loop(0, half, body, 0, unroll=8)

        def drain_and_select(s):
            k_, t = divmod(s, 3)
            tab, smem = tabs[t], smems[t]
            buf = tbufs[s % 2]
            sa, sb = sems[s % 2]
            pltpu.make_async_copy(
                tab.at[pl.ds(0, half)], buf.at[pl.ds(0, half)], sa).wait()
            pltpu.make_async_copy(
                tab.at[pl.ds(0, half)], buf.at[pl.ds(half, half)], sb).wait()
            off = k_ * CHUNK

            def body(i, carry):
                j = smem[off + i] & 7
                for cg in range(D // 16):
                    csl = pl.ds(cg * 16, 16)
                    asm[i, pl.ds(t * D + cg * 16, 16)] = buf[i, j, csl]
                return carry

            lax.fori_loop(0, CHUNK, body, 0)

        nsteps = nch * 3
        fire(0)
        for s in range(nsteps):
            if s + 1 < nsteps:
                fire(s + 1)
            drain_and_select(s)
            k_, t = divmod(s, 3)
            if t == 2:
                pltpu.sync_copy(
                    asm, out_h.at[pl.ds(base + k_ * CHUNK, CHUNK)])

    return k(uid, rid, ing, ut3, rt3, it3)


def kernel(uid, rid, ing, user_table, recipe_table, ingredient_table):
    ut3 = user_table.reshape(-1, 8, D)
    rt3 = recipe_table.reshape(-1, 8, D)
    it3 = ingredient_table.reshape(-1, 8, D)
    return _gather_concat(uid, rid, ing, ut3, rt3, it3)


# speedup vs baseline: 1.0467x; 1.0348x over previous
import functools

import jax
import jax.numpy as jnp
from jax import lax
from jax.experimental import pallas as pl
from jax.experimental.pallas import tpu as pltpu
from jax.experimental.pallas import tpu_sc as plsc

B = 16384
D = 64
CHUNK = 32


def _gather_concat(uid, rid2, ridp, ing2, ingp, ut3, rt2, it2):
    info = plsc.get_sparse_core_info()
    nc, ns = info.num_cores, info.num_subcores
    nw = nc * ns
    bpw = B // nw
    nch = bpw // CHUNK
    mesh = plsc.VectorSubcoreMesh(core_axis_name="c", subcore_axis_name="s")

    @functools.partial(
        pl.kernel,
        mesh=mesh,
        out_type=jax.ShapeDtypeStruct((B, 3 * D), jnp.float32),
        scratch_types=[
            pltpu.SMEM((bpw,), jnp.int32),
            pltpu.SMEM((bpw,), jnp.int32),
            pltpu.SMEM((bpw,), jnp.int32),
            pltpu.VMEM_SHARED((B,), jnp.int32),
            pltpu.VMEM((bpw,), jnp.int32),
            pltpu.VMEM((bpw,), jnp.int32),
            pltpu.VMEM((CHUNK, 8, D), jnp.float32),
            pltpu.VMEM((CHUNK, 8, D), jnp.float32),
            pltpu.VMEM((CHUNK, 2 * D), jnp.float32),
            pltpu.VMEM((CHUNK, 2 * D), jnp.float32),
            pltpu.VMEM((CHUNK, 2 * D), jnp.float32),
            pltpu.VMEM((CHUNK, 2 * D), jnp.float32),
            pltpu.VMEM((CHUNK, 3 * D), jnp.float32),
            pltpu.SemaphoreType.DMA,
            pltpu.SemaphoreType.DMA,
            pltpu.SemaphoreType.DMA,
            pltpu.SemaphoreType.DMA,
        ],
    )
    def k(uid_h, rid2_h, ridp_h, ing2_h, ingp_h, ut_h, rt_h, it_h, out_h,
          smem_u, smem_rp, smem_gp, sp, rix, gix,
          ub0, ub1, rb0, rb1, gb0, gb1, asm,
          semu0, semu1, semrg0, semrg1):
        wid = lax.axis_index("s") * nc + lax.axis_index("c")
        base = wid * bpw
        bsl = pl.ds(base, bpw)

        for ix_h, smem in ((uid_h, smem_u), (ridp_h, smem_rp),
                           (ingp_h, smem_gp)):
            pltpu.sync_copy(ix_h.at[bsl], sp.at[bsl])
            pltpu.sync_copy(sp.at[bsl], smem)
        pltpu.sync_copy(rid2_h.at[bsl], rix)
        pltpu.sync_copy(ing2_h.at[bsl], gix)

        ubufs = (ub0, ub1)
        rbufs = (rb0, rb1)
        gbufs = (gb0, gb1)
        usems = (semu0, semu1)
        rgsems = (semrg0, semrg1)

        def fire(k_):
            par = k_ % 2
            off = k_ * CHUNK
            sl = pl.ds(off, CHUNK)
            srg = rgsems[par]
            pltpu.async_copy(rt_h.at[rix.at[sl]], rbufs[par], srg)
            pltpu.async_copy(it_h.at[gix.at[sl]], gbufs[par], srg)
            ubuf, su = ubufs[par], usems[par]

            def body(i, carry):
                blk = smem_u[off + i] >> 3
                pltpu.async_copy(ut_h.at[blk], ubuf.at[i], su)
                return carry

            lax.fori_loop(0, CHUNK, body, 0, unroll=8)

        def drain_and_select(k_):
            par = k_ % 2
            off = k_ * CHUNK
            ubuf, rbuf, gbuf = ubufs[par], rbufs[par], gbufs[par]
            pltpu.make_async_copy(
                ut_h.at[pl.ds(0, CHUNK)], ubuf, usems[par]).wait()
            srg = rgsems[par]
            pltpu.make_async_copy(
                rt_h.at[pl.ds(0, CHUNK)], rbuf, srg).wait()
            pltpu.make_async_copy(
                it_h.at[pl.ds(0, CHUNK)], gbuf, srg).wait()

            def body(i, carry):
                j = smem_u[off + i] & 7
                rp = smem_rp[off + i]
                gp = smem_gp[off + i]
                for cg in range(D // 16):
                    csl = pl.ds(cg * 16, 16)
                    asm[i, pl.ds(cg * 16, 16)] = ubuf[i, j, csl]

                @pl.when(rp == 0)
                def _():
                    for cg in range(D // 16):
                        asm[i, pl.ds(D + cg * 16, 16)] = \
                            rbuf[i, pl.ds(cg * 16, 16)]

                @pl.when(rp == 1)
                def _():
                    for cg in range(D // 16):
                        asm[i, pl.ds(D + cg * 16, 16)] = \
                            rbuf[i, pl.ds(D + cg * 16, 16)]

                @pl.when(gp == 0)
                def _():
                    for cg in range(D // 16):
                        asm[i, pl.ds(2 * D + cg * 16, 16)] = \
                            gbuf[i, pl.ds(cg * 16, 16)]

                @pl.when(gp == 1)
                def _():
                    for cg in range(D // 16):
                        asm[i, pl.ds(2 * D + cg * 16, 16)] = \
                            gbuf[i, pl.ds(D + cg * 16, 16)]

                return carry

            lax.fori_loop(0, CHUNK, body, 0)

        fire(0)
        for k_ in range(nch):
            if k_ + 1 < nch:
                fire(k_ + 1)
            drain_and_select(k_)
            pltpu.sync_copy(asm, out_h.at[pl.ds(base + k_ * CHUNK, CHUNK)])

    return k(uid, rid2, ridp, ing2, ingp, ut3, rt2, it2)


def kernel(uid, rid, ing, user_table, recipe_table, ingredient_table):
    ut3 = user_table.reshape(-1, 8, D)
    rt2 = recipe_table.reshape(-1, 2 * D)
    it2 = ingredient_table.reshape(-1, 2 * D)
    return _gather_concat(
        uid, rid >> 1, rid & 1, ing >> 1, ing & 1, ut3, rt2, it2)
